# Initial kernel scaffold; baseline (speedup 1.0000x reference)
import jax
import jax.numpy as jnp
from jax.experimental import pallas as pl


def kernel(x, edge_index, edge_attr, batch, params):
    raise NotImplementedError("write your pallas kernel here")


# speedup vs baseline: 11.1924x; 11.1924x over previous
import functools

import jax
import jax.numpy as jnp
from jax import lax
from jax.experimental import pallas as pl
from jax.experimental.pallas import tpu as pltpu
from jax.experimental.pallas import tpu_sc as plsc

N_NODES = 10000
N_EDGES = 320000
N_GRAPHS = 16
N_LAYERS = 5
NSUP = 16
NOUT = 16
PDIM = NSUP * NOUT

NC = 2
NS = 16
NW = NC * NS
EPW = N_EDGES // NW
CHUNK = 80
NB = CHUNK * NOUT // 128
NCHUNK = EPW // CHUNK
ACC_SLICE = 640
N_PAD = ACC_SLICE * NS
FLAT = N_PAD * NOUT

def _sc_spectconv_body(p_hbm, ea_hbm, src_hbm, idx_hbm, zero_hbm, out_hbm,
                       src_v, idx_v, rows_v, ea_v, msgf_v, acc_sh, sem):
    cid = lax.axis_index("c")
    sid = lax.axis_index("s")
    wid = cid * NS + sid
    f0 = sid * (FLAT // NS)
    pltpu.sync_copy(zero_hbm, acc_sh.at[pl.ds(f0, FLAT // NS)])
    plsc.subcore_barrier()

    def chunk_body(ci, carry):
        base = wid * EPW + ci * CHUNK
        pltpu.sync_copy(src_hbm.at[pl.ds(base, CHUNK)], src_v)
        pltpu.sync_copy(idx_hbm.at[wid, ci], idx_v)
        pltpu.sync_copy(ea_hbm.at[pl.ds(base, CHUNK)], ea_v)
        pltpu.async_copy(p_hbm.at[src_v], rows_v, sem).wait()

        def edge_body(e, c2):
            av = ea_v[e, :]
            m = av[0] * rows_v[e, pl.ds(0, NOUT)]
            for i in range(1, NSUP):
                m = m + av[i] * rows_v[e, pl.ds(i * NOUT, NOUT)]
            msgf_v[pl.ds(e * NOUT, NOUT)] = m
            return c2

        lax.fori_loop(0, CHUNK, edge_body, 0)

        def scat_body(b, c2):
            pltpu.sync_copy(msgf_v.at[pl.ds(b * 128, 128)],
                            acc_sh.at[idx_v.at[b]], add=True)
            return c2

        lax.fori_loop(0, NB, scat_body, 0)
        return carry

    lax.fori_loop(0, NCHUNK, chunk_body, 0)
    plsc.subcore_barrier()
    pltpu.sync_copy(acc_sh.at[pl.ds(f0, FLAT // NS)],
                    out_hbm.at[cid, pl.ds(f0, FLAT // NS)])


@functools.cache
def _sc_spectconv():
    mesh = plsc.VectorSubcoreMesh(
        core_axis_name="c", subcore_axis_name="s",
        num_cores=NC, num_subcores=NS)
    return pl.kernel(
        _sc_spectconv_body,
        out_type=jax.ShapeDtypeStruct((NC, FLAT), jnp.float32),
        mesh=mesh,
        scratch_types=[
            pltpu.VMEM((CHUNK,), jnp.int32),
            pltpu.VMEM((NB, 128), jnp.int32),
            pltpu.VMEM((CHUNK, PDIM), jnp.float32),
            pltpu.VMEM((CHUNK, NSUP), jnp.float32),
            pltpu.VMEM((CHUNK * NOUT,), jnp.float32),
            pltpu.VMEM_SHARED((FLAT,), jnp.float32),
            pltpu.SemaphoreType.DMA,
        ],
    )


def _edge_mlp_body(eattr_ref, dstr_ref, wabc_ref, w14_ref, out_ref, idx_ref):
    l = pl.program_id(1)
    ea = eattr_ref[...]
    t = jnp.dot(ea, wabc_ref[0], preferred_element_type=jnp.float32)
    tmp = (jax.nn.relu(t[:, :32])
           + jax.nn.relu(t[:, 32:64]) * jax.nn.relu(t[:, 64:96]))
    out_ref[0] = jax.nn.relu(
        jnp.dot(tmp, w14_ref[0], preferred_element_type=jnp.float32))

    @pl.when(l == 0)
    def _():
        sel = (lax.broadcasted_iota(jnp.int32, (8, 128), 1) // NOUT
               == lax.broadcasted_iota(jnp.int32, (8, 128), 0)).astype(jnp.float32)
        idxf = jnp.dot(dstr_ref[0], sel, preferred_element_type=jnp.float32,
                       precision=lax.Precision.HIGHEST)
        lanes = lax.broadcasted_iota(jnp.int32, (EPW // 8, 128), 1) % NOUT
        idx = (idxf * NOUT).astype(jnp.int32) + lanes
        idx_ref[...] = idx.reshape(1, NCHUNK, NB, 128)


def _edge_mlp(edge_attr, dstr, wabc, w14):
    return pl.pallas_call(
        _edge_mlp_body,
        grid=(NW, N_LAYERS),
        in_specs=[
            pl.BlockSpec((EPW, NSUP), lambda i, l: (i, 0)),
            pl.BlockSpec((1, EPW // 8, 8), lambda i, l: (i, 0, 0)),
            pl.BlockSpec((1, NSUP, 96), lambda i, l: (l, 0, 0)),
            pl.BlockSpec((1, 32, NSUP), lambda i, l: (l, 0, 0)),
        ],
        out_specs=[
            pl.BlockSpec((1, EPW, NSUP), lambda i, l: (l, i, 0)),
            pl.BlockSpec((1, NCHUNK, NB, 128), lambda i, l: (i, 0, 0, 0)),
        ],
        out_shape=[
            jax.ShapeDtypeStruct((N_LAYERS, N_EDGES, NSUP), jnp.float32),
            jax.ShapeDtypeStruct((NW, NCHUNK, NB, 128), jnp.int32),
        ],
    )(edge_attr, dstr, wabc, w14)


def _proj0_body(x_ref, wcf_ref, wf1_ref, wf2_ref, bf1_ref, bf2_ref,
                p_ref, g_ref):
    x = x_ref[...]
    p_ref[...] = jnp.dot(x, wcf_ref[...], preferred_element_type=jnp.float32)
    g_ref[...] = (
        jnp.tanh(jnp.dot(x, wf1_ref[...], preferred_element_type=jnp.float32)
                 + bf1_ref[...])
        * jnp.tanh(jnp.dot(x, wf2_ref[...], preferred_element_type=jnp.float32)
                   + bf2_ref[...]))


def _proj0(x, wcf, wf1, wf2, bf1, bf2):
    return pl.pallas_call(
        _proj0_body,
        out_shape=[
            jax.ShapeDtypeStruct((N_NODES, PDIM), jnp.float32),
            jax.ShapeDtypeStruct((N_NODES, NOUT), jnp.float32),
        ],
    )(x, wcf, wf1, wf2, bf1, bf2)


def _proj_mid_body(part_ref, gate_ref, bc_ref, wcf_ref, wf1_ref, wf2_ref,
                   bf1_ref, bf2_ref, p_ref, g_ref):
    out = jax.nn.relu(part_ref[0] + part_ref[1] + bc_ref[...])
    h = jnp.concatenate([out, gate_ref[...]], axis=1)
    p_ref[...] = jnp.dot(h, wcf_ref[...], preferred_element_type=jnp.float32)
    g_ref[...] = (
        jnp.tanh(jnp.dot(h, wf1_ref[...], preferred_element_type=jnp.float32)
                 + bf1_ref[...])
        * jnp.tanh(jnp.dot(h, wf2_ref[...], preferred_element_type=jnp.float32)
                   + bf2_ref[...]))


def _proj_mid(parts, gate, bc, wcf, wf1, wf2, bf1, bf2):
    return pl.pallas_call(
        _proj_mid_body,
        out_shape=[
            jax.ShapeDtypeStruct((N_NODES, PDIM), jnp.float32),
            jax.ShapeDtypeStruct((N_NODES, NOUT), jnp.float32),
        ],
    )(parts, gate, bc, wcf, wf1, wf2, bf1, bf2)


def _final_body(part_ref, gate_ref, bc_ref, batch_ref, fc1w_ref, fc1b_ref,
                fc2w_ref, fc2b_ref, out_ref):
    out = jax.nn.relu(part_ref[0] + part_ref[1] + bc_ref[...])
    h = jnp.concatenate([out, gate_ref[...]], axis=1)
    iota = lax.broadcasted_iota(jnp.int32, (N_GRAPHS, N_NODES), 0)
    onehot = jnp.where(iota == batch_ref[...], 1.0, 0.0)
    pooled = jnp.dot(onehot, h, preferred_element_type=jnp.float32)
    z = jax.nn.relu(
        jnp.dot(pooled, fc1w_ref[...], preferred_element_type=jnp.float32)
        + fc1b_ref[...])
    out_ref[...] = (jnp.dot(z, fc2w_ref[...],
                            preferred_element_type=jnp.float32)
                    + fc2b_ref[...])


def _final(parts, gate, bc, batch2d, fc1w, fc1b, fc2w, fc2b):
    return pl.pallas_call(
        _final_body,
        out_shape=jax.ShapeDtypeStruct((N_GRAPHS, 1), jnp.float32),
    )(parts, gate, bc, batch2d, fc1w, fc1b, fc2w, fc2b)


def kernel(x, edge_index, edge_attr, batch, params):
    layers = params["layers"]
    src = edge_index[0]
    dstr = edge_index[1].reshape(NW, EPW // 8, 8).astype(jnp.float32)

    wabc = jnp.stack([
        jnp.concatenate([lp["W11"], lp["W12"], lp["W13"]], axis=1)
        for lp in layers])
    w14 = jnp.stack([lp["W14"] for lp in layers])
    wcf = [jnp.transpose(lp["Wc"], (1, 0, 2)).reshape(-1, PDIM)
           for lp in layers]
    zero_hbm = jnp.zeros((FLAT // NS,), jnp.float32)

    ea_all, idx4 = _edge_mlp(edge_attr, dstr, wabc, w14)

    lp = layers[0]
    p, gate = _proj0(x, wcf[0],
                     lp["Wf11"], lp["Wf12"],
                     lp["bf11"].reshape(1, -1), lp["bf12"].reshape(1, -1))
    for l in range(N_LAYERS):
        parts = _sc_spectconv()(p, ea_all[l], src, idx4, zero_hbm)
        parts = parts.reshape(NC, N_PAD, NOUT)[:, :N_NODES, :]
        if l + 1 < N_LAYERS:
            nxt = layers[l + 1]
            p, gate = _proj_mid(parts, gate,
                                layers[l]["bc"].reshape(1, -1), wcf[l + 1],
                                nxt["Wf11"], nxt["Wf12"],
                                nxt["bf11"].reshape(1, -1),
                                nxt["bf12"].reshape(1, -1))
        else:
            out = _final(parts, gate, layers[l]["bc"].reshape(1, -1),
                         batch.reshape(1, -1).astype(jnp.int32),
                         params["fc1_W"], params["fc1_b"].reshape(1, -1),
                         params["fc2_W"], params["fc2_b"].reshape(1, -1))
    return out


# speedup vs baseline: 16.8757x; 1.5078x over previous
import functools

import jax
import jax.numpy as jnp
from jax import lax
from jax.experimental import pallas as pl
from jax.experimental.pallas import tpu as pltpu
from jax.experimental.pallas import tpu_sc as plsc

N_NODES = 10000
N_EDGES = 320000
N_GRAPHS = 16
N_LAYERS = 5
NSUP = 16
NOUT = 16
PDIM = NSUP * NOUT

NC = 2
NS = 16
NW = NC * NS
EPW = N_EDGES // NW
CHUNK = 80
NB = CHUNK * NOUT // 128
NCHUNK = EPW // CHUNK
ACC_SLICE = 640
N_PAD = ACC_SLICE * NS
FLAT = N_PAD * NOUT

def _sc_spectconv_body(p_hbm, ea_hbm, src_hbm, idx_hbm, zero_hbm, out_hbm,
                       src_v, idx_v, rows_v, ea_v, msgf_v, acc_sh,
                       lsem, gsem, ssem):
    cid = lax.axis_index("c")
    sid = lax.axis_index("s")
    wid = cid * NS + sid
    f0 = sid * (FLAT // NS)
    pltpu.sync_copy(zero_hbm, acc_sh.at[pl.ds(f0, FLAT // NS)])
    plsc.subcore_barrier()

    def fire_linear(ci, s):
        base = wid * EPW + ci * CHUNK
        pltpu.async_copy(src_hbm.at[pl.ds(base, CHUNK)], src_v[s], lsem[s])
        pltpu.async_copy(ea_hbm.at[pl.ds(base, CHUNK)], ea_v[s], lsem[s])
        pltpu.async_copy(idx_hbm.at[wid, ci], idx_v[s], lsem[s])

    def drain_linear(s):
        pltpu.make_async_copy(src_hbm.at[pl.ds(0, CHUNK)], src_v[s],
                              lsem[s]).wait()
        pltpu.make_async_copy(ea_hbm.at[pl.ds(0, CHUNK)], ea_v[s],
                              lsem[s]).wait()
        pltpu.make_async_copy(idx_hbm.at[0, 0], idx_v[s], lsem[s]).wait()

    def fire_gather(s):
        pltpu.async_copy(p_hbm.at[src_v[s]], rows_v[s], gsem[s])

    def drain_gather(s):
        pltpu.make_async_copy(p_hbm.at[src_v[s]], rows_v[s], gsem[s]).wait()

    def fire_scatter(s):
        descs = [pltpu.async_copy(msgf_v[s].at[pl.ds(b * 128, 128)],
                                  acc_sh.at[idx_v[s].at[b]], ssem[s], add=True)
                 for b in range(NB)]
        for d in descs:
            d.wait()

    def drain_scatter(s):
        pass

    def compute(s):
        def edge_body(e, c2):
            av = ea_v[s][e, :]
            m = av[0] * rows_v[s][e, pl.ds(0, NOUT)]
            for i in range(1, NSUP):
                m = m + av[i] * rows_v[s][e, pl.ds(i * NOUT, NOUT)]
            msgf_v[s][pl.ds(e * NOUT, NOUT)] = m
            return c2
        lax.fori_loop(0, CHUNK, edge_body, 0, unroll=2)

    fire_linear(0, 0)
    drain_linear(0)
    fire_gather(0)

    def pair_body(g, carry):
        for s in range(2):
            ci = 2 * g + s
            if s == 0:
                @pl.when(g > 0)
                def _():
                    drain_scatter(1)
                    fire_linear(ci + 1, 1)
                    drain_linear(1)
                    drain_gather(0)
                    fire_gather(1)

                @pl.when(g == 0)
                def _():
                    fire_linear(ci + 1, 1)
                    drain_linear(1)
                    drain_gather(0)
                    fire_gather(1)
            else:
                drain_scatter(0)

                @pl.when(g < NCHUNK // 2 - 1)
                def _():
                    fire_linear(ci + 1, 0)
                    drain_linear(0)
                    drain_gather(1)
                    fire_gather(0)

                @pl.when(g == NCHUNK // 2 - 1)
                def _():
                    drain_gather(1)
            compute(s)
            fire_scatter(s)
        return carry

    lax.fori_loop(0, NCHUNK // 2, pair_body, 0)
    fire_linear(NCHUNK - 1, 0)
    drain_linear(0)
    fire_gather(0)
    drain_gather(0)
    compute(0)
    fire_scatter(0)
    drain_scatter(1)
    drain_scatter(0)
    plsc.subcore_barrier()
    pltpu.sync_copy(acc_sh.at[pl.ds(f0, FLAT // NS)],
                    out_hbm.at[cid, pl.ds(f0, FLAT // NS)])


@functools.cache
def _sc_spectconv():
    mesh = plsc.VectorSubcoreMesh(
        core_axis_name="c", subcore_axis_name="s",
        num_cores=NC, num_subcores=NS)
    return pl.kernel(
        _sc_spectconv_body,
        out_type=jax.ShapeDtypeStruct((NC, FLAT), jnp.float32),
        mesh=mesh,
        scratch_types=[
            [pltpu.VMEM((CHUNK,), jnp.int32)] * 2,
            [pltpu.VMEM((NB, 128), jnp.int32)] * 2,
            [pltpu.VMEM((CHUNK, PDIM), jnp.float32)] * 2,
            [pltpu.VMEM((CHUNK, NSUP), jnp.float32)] * 2,
            [pltpu.VMEM((CHUNK * NOUT,), jnp.float32)] * 2,
            pltpu.VMEM_SHARED((FLAT,), jnp.float32),
            [pltpu.SemaphoreType.DMA] * 2,
            [pltpu.SemaphoreType.DMA] * 2,
            [pltpu.SemaphoreType.DMA] * 2,
        ],
    )


def _edge_mlp_body(eattr_ref, dstr_ref, wabc_ref, w14_ref, out_ref, idx_ref):
    l = pl.program_id(1)
    ea = eattr_ref[...]
    t = jnp.dot(ea, wabc_ref[0], preferred_element_type=jnp.float32)
    tmp = (jax.nn.relu(t[:, :32])
           + jax.nn.relu(t[:, 32:64]) * jax.nn.relu(t[:, 64:96]))
    out_ref[0] = jax.nn.relu(
        jnp.dot(tmp, w14_ref[0], preferred_element_type=jnp.float32))

    @pl.when(l == 0)
    def _():
        sel = (lax.broadcasted_iota(jnp.int32, (8, 128), 1) // NOUT
               == lax.broadcasted_iota(jnp.int32, (8, 128), 0)).astype(jnp.float32)
        idxf = jnp.dot(dstr_ref[0], sel, preferred_element_type=jnp.float32,
                       precision=lax.Precision.HIGHEST)
        lanes = lax.broadcasted_iota(jnp.int32, (EPW // 8, 128), 1) % NOUT
        idx = (idxf * NOUT).astype(jnp.int32) + lanes
        idx_ref[...] = idx.reshape(1, NCHUNK, NB, 128)


def _edge_mlp(edge_attr, dstr, wabc, w14):
    return pl.pallas_call(
        _edge_mlp_body,
        grid=(NW, N_LAYERS),
        in_specs=[
            pl.BlockSpec((EPW, NSUP), lambda i, l: (i, 0)),
            pl.BlockSpec((1, EPW // 8, 8), lambda i, l: (i, 0, 0)),
            pl.BlockSpec((1, NSUP, 96), lambda i, l: (l, 0, 0)),
            pl.BlockSpec((1, 32, NSUP), lambda i, l: (l, 0, 0)),
        ],
        out_specs=[
            pl.BlockSpec((1, EPW, NSUP), lambda i, l: (l, i, 0)),
            pl.BlockSpec((1, NCHUNK, NB, 128), lambda i, l: (i, 0, 0, 0)),
        ],
        out_shape=[
            jax.ShapeDtypeStruct((N_LAYERS, N_EDGES, NSUP), jnp.float32),
            jax.ShapeDtypeStruct((NW, NCHUNK, NB, 128), jnp.int32),
        ],
    )(edge_attr, dstr, wabc, w14)


def _proj0_body(x_ref, wcf_ref, wf1_ref, wf2_ref, bf1_ref, bf2_ref,
                p_ref, g_ref):
    x = x_ref[...]
    p_ref[...] = jnp.dot(x, wcf_ref[...], preferred_element_type=jnp.float32)
    g_ref[...] = (
        jnp.tanh(jnp.dot(x, wf1_ref[...], preferred_element_type=jnp.float32)
                 + bf1_ref[...])
        * jnp.tanh(jnp.dot(x, wf2_ref[...], preferred_element_type=jnp.float32)
                   + bf2_ref[...]))


def _proj0(x, wcf, wf1, wf2, bf1, bf2):
    return pl.pallas_call(
        _proj0_body,
        out_shape=[
            jax.ShapeDtypeStruct((N_NODES, PDIM), jnp.float32),
            jax.ShapeDtypeStruct((N_NODES, NOUT), jnp.float32),
        ],
    )(x, wcf, wf1, wf2, bf1, bf2)


def _proj_mid_body(part_ref, gate_ref, bc_ref, wcf_ref, wf1_ref, wf2_ref,
                   bf1_ref, bf2_ref, p_ref, g_ref):
    out = jax.nn.relu(part_ref[0] + part_ref[1] + bc_ref[...])
    h = jnp.concatenate([out, gate_ref[...]], axis=1)
    p_ref[...] = jnp.dot(h, wcf_ref[...], preferred_element_type=jnp.float32)
    g_ref[...] = (
        jnp.tanh(jnp.dot(h, wf1_ref[...], preferred_element_type=jnp.float32)
                 + bf1_ref[...])
        * jnp.tanh(jnp.dot(h, wf2_ref[...], preferred_element_type=jnp.float32)
                   + bf2_ref[...]))


def _proj_mid(parts, gate, bc, wcf, wf1, wf2, bf1, bf2):
    return pl.pallas_call(
        _proj_mid_body,
        out_shape=[
            jax.ShapeDtypeStruct((N_NODES, PDIM), jnp.float32),
            jax.ShapeDtypeStruct((N_NODES, NOUT), jnp.float32),
        ],
    )(parts, gate, bc, wcf, wf1, wf2, bf1, bf2)


def _final_body(part_ref, gate_ref, bc_ref, batch_ref, fc1w_ref, fc1b_ref,
                fc2w_ref, fc2b_ref, out_ref):
    out = jax.nn.relu(part_ref[0] + part_ref[1] + bc_ref[...])
    h = jnp.concatenate([out, gate_ref[...]], axis=1)
    iota = lax.broadcasted_iota(jnp.int32, (N_GRAPHS, N_NODES), 0)
    onehot = jnp.where(iota == batch_ref[...], 1.0, 0.0)
    pooled = jnp.dot(onehot, h, preferred_element_type=jnp.float32)
    z = jax.nn.relu(
        jnp.dot(pooled, fc1w_ref[...], preferred_element_type=jnp.float32)
        + fc1b_ref[...])
    out_ref[...] = (jnp.dot(z, fc2w_ref[...],
                            preferred_element_type=jnp.float32)
                    + fc2b_ref[...])


def _final(parts, gate, bc, batch2d, fc1w, fc1b, fc2w, fc2b):
    return pl.pallas_call(
        _final_body,
        out_shape=jax.ShapeDtypeStruct((N_GRAPHS, 1), jnp.float32),
    )(parts, gate, bc, batch2d, fc1w, fc1b, fc2w, fc2b)


def kernel(x, edge_index, edge_attr, batch, params):
    layers = params["layers"]
    src = edge_index[0]
    dstr = edge_index[1].reshape(NW, EPW // 8, 8).astype(jnp.float32)

    wabc = jnp.stack([
        jnp.concatenate([lp["W11"], lp["W12"], lp["W13"]], axis=1)
        for lp in layers])
    w14 = jnp.stack([lp["W14"] for lp in layers])
    wcf = [jnp.transpose(lp["Wc"], (1, 0, 2)).reshape(-1, PDIM)
           for lp in layers]
    zero_hbm = jnp.zeros((FLAT // NS,), jnp.float32)

    ea_all, idx4 = _edge_mlp(edge_attr, dstr, wabc, w14)

    lp = layers[0]
    p, gate = _proj0(x, wcf[0],
                     lp["Wf11"], lp["Wf12"],
                     lp["bf11"].reshape(1, -1), lp["bf12"].reshape(1, -1))
    for l in range(N_LAYERS):
        parts = _sc_spectconv()(p, ea_all[l], src, idx4, zero_hbm)
        parts = parts.reshape(NC, N_PAD, NOUT)[:, :N_NODES, :]
        if l + 1 < N_LAYERS:
            nxt = layers[l + 1]
            p, gate = _proj_mid(parts, gate,
                                layers[l]["bc"].reshape(1, -1), wcf[l + 1],
                                nxt["Wf11"], nxt["Wf12"],
                                nxt["bf11"].reshape(1, -1),
                                nxt["bf12"].reshape(1, -1))
        else:
            out = _final(parts, gate, layers[l]["bc"].reshape(1, -1),
                         batch.reshape(1, -1).astype(jnp.int32),
                         params["fc1_W"], params["fc1_b"].reshape(1, -1),
                         params["fc2_W"], params["fc2_b"].reshape(1, -1))
    return out


# speedup vs baseline: 19.6271x; 1.1630x over previous
import functools

import jax
import jax.numpy as jnp
from jax import lax
from jax.experimental import pallas as pl
from jax.experimental.pallas import tpu as pltpu
from jax.experimental.pallas import tpu_sc as plsc

N_NODES = 10000
N_EDGES = 320000
N_GRAPHS = 16
N_LAYERS = 5
NSUP = 16
NOUT = 16
PDIM = NSUP * NOUT

NC = 2
NS = 16
NW = NC * NS
EPW = N_EDGES // NW
CHUNK = 80
NB = CHUNK * NOUT // 128
NCHUNK = EPW // CHUNK
ACC_SLICE = 640
N_PAD = ACC_SLICE * NS
FLAT = N_PAD * NOUT

def _sc_spectconv_body(layer, p_hbm, ea_hbm, src_hbm, idx_hbm, zero_hbm,
                       out_hbm, src_v, idx_v, rows_v, ea_v, msgf_v, acc_sh,
                       lsem, gsem, ssem):
    cid = lax.axis_index("c")
    sid = lax.axis_index("s")
    wid = cid * NS + sid
    f0 = sid * (FLAT // NS)
    pltpu.sync_copy(zero_hbm, acc_sh.at[pl.ds(f0, FLAT // NS)])
    plsc.subcore_barrier()

    def fire_linear(ci, s):
        base = wid * EPW + ci * CHUNK
        pltpu.async_copy(src_hbm.at[pl.ds(base, CHUNK)], src_v[s], lsem[s])
        pltpu.async_copy(ea_hbm.at[layer, pl.ds(base, CHUNK)], ea_v[s],
                         lsem[s])
        pltpu.async_copy(idx_hbm.at[wid, ci], idx_v[s], lsem[s])

    def drain_linear(s):
        pltpu.make_async_copy(src_hbm.at[pl.ds(0, CHUNK)], src_v[s],
                              lsem[s]).wait()
        pltpu.make_async_copy(ea_hbm.at[0, pl.ds(0, CHUNK)], ea_v[s],
                              lsem[s]).wait()
        pltpu.make_async_copy(idx_hbm.at[0, 0], idx_v[s], lsem[s]).wait()

    def fire_gather(s):
        pltpu.async_copy(p_hbm.at[src_v[s]], rows_v[s], gsem[s])

    def drain_gather(s):
        pltpu.make_async_copy(p_hbm.at[src_v[s]], rows_v[s], gsem[s]).wait()

    def fire_scatter(s):
        descs = [pltpu.async_copy(msgf_v[s].at[pl.ds(b * 128, 128)],
                                  acc_sh.at[idx_v[s].at[b]], ssem[s], add=True)
                 for b in range(NB)]
        for d in descs:
            d.wait()

    def drain_scatter(s):
        pass

    def compute(s):
        def edge_body(e, c2):
            av = ea_v[s][e, :]
            m = av[0] * rows_v[s][e, pl.ds(0, NOUT)]
            for i in range(1, NSUP):
                m = m + av[i] * rows_v[s][e, pl.ds(i * NOUT, NOUT)]
            msgf_v[s][pl.ds(e * NOUT, NOUT)] = m
            return c2
        lax.fori_loop(0, CHUNK, edge_body, 0, unroll=2)

    fire_linear(0, 0)
    drain_linear(0)
    fire_gather(0)

    def pair_body(g, carry):
        for s in range(2):
            ci = 2 * g + s
            if s == 0:
                @pl.when(g > 0)
                def _():
                    drain_scatter(1)
                    fire_linear(ci + 1, 1)
                    drain_linear(1)
                    drain_gather(0)
                    fire_gather(1)

                @pl.when(g == 0)
                def _():
                    fire_linear(ci + 1, 1)
                    drain_linear(1)
                    drain_gather(0)
                    fire_gather(1)
            else:
                drain_scatter(0)

                @pl.when(g < NCHUNK // 2 - 1)
                def _():
                    fire_linear(ci + 1, 0)
                    drain_linear(0)
                    drain_gather(1)
                    fire_gather(0)

                @pl.when(g == NCHUNK // 2 - 1)
                def _():
                    drain_gather(1)
            compute(s)
            fire_scatter(s)
        return carry

    lax.fori_loop(0, NCHUNK // 2, pair_body, 0)
    fire_linear(NCHUNK - 1, 0)
    drain_linear(0)
    fire_gather(0)
    drain_gather(0)
    compute(0)
    fire_scatter(0)
    drain_scatter(1)
    drain_scatter(0)
    plsc.subcore_barrier()
    pltpu.sync_copy(acc_sh.at[pl.ds(f0, FLAT // NS)],
                    out_hbm.at[cid, pl.ds(f0, FLAT // NS)])


@functools.cache
def _sc_spectconv(layer):
    mesh = plsc.VectorSubcoreMesh(
        core_axis_name="c", subcore_axis_name="s",
        num_cores=NC, num_subcores=NS)
    return pl.kernel(
        functools.partial(_sc_spectconv_body, layer),
        out_type=jax.ShapeDtypeStruct((NC, FLAT), jnp.float32),
        mesh=mesh,
        scratch_types=[
            [pltpu.VMEM((CHUNK,), jnp.int32)] * 2,
            [pltpu.VMEM((NB, 128), jnp.int32)] * 2,
            [pltpu.VMEM((CHUNK, PDIM), jnp.float32)] * 2,
            [pltpu.VMEM((CHUNK, NSUP), jnp.float32)] * 2,
            [pltpu.VMEM((CHUNK * NOUT,), jnp.float32)] * 2,
            pltpu.VMEM_SHARED((FLAT,), jnp.float32),
            [pltpu.SemaphoreType.DMA] * 2,
            [pltpu.SemaphoreType.DMA] * 2,
            [pltpu.SemaphoreType.DMA] * 2,
        ],
    )


def _edge_mlp_body(eattr_ref, dstr_ref, wabc_ref, w14_ref, out_ref, idx_ref):
    l = pl.program_id(1)
    ea = eattr_ref[...]
    t = jnp.dot(ea, wabc_ref[0], preferred_element_type=jnp.float32)
    tmp = (jax.nn.relu(t[:, :32])
           + jax.nn.relu(t[:, 32:64]) * jax.nn.relu(t[:, 64:96]))
    out_ref[0] = jax.nn.relu(
        jnp.dot(tmp, w14_ref[0], preferred_element_type=jnp.float32))

    @pl.when(l == 0)
    def _():
        sel = (lax.broadcasted_iota(jnp.int32, (8, 128), 1) // NOUT
               == lax.broadcasted_iota(jnp.int32, (8, 128), 0)).astype(jnp.float32)
        idxf = jnp.dot(dstr_ref[0], sel, preferred_element_type=jnp.float32,
                       precision=lax.Precision.HIGHEST)
        lanes = lax.broadcasted_iota(jnp.int32, (EPW // 8, 128), 1) % NOUT
        idx = (idxf * NOUT).astype(jnp.int32) + lanes
        idx_ref[...] = idx.reshape(1, NCHUNK, NB, 128)


def _edge_mlp(edge_attr, dstr, wabc, w14):
    return pl.pallas_call(
        _edge_mlp_body,
        grid=(NW, N_LAYERS),
        in_specs=[
            pl.BlockSpec((EPW, NSUP), lambda i, l: (i, 0)),
            pl.BlockSpec((1, EPW // 8, 8), lambda i, l: (i, 0, 0)),
            pl.BlockSpec((1, NSUP, 96), lambda i, l: (l, 0, 0)),
            pl.BlockSpec((1, 32, NSUP), lambda i, l: (l, 0, 0)),
        ],
        out_specs=[
            pl.BlockSpec((1, EPW, NSUP), lambda i, l: (l, i, 0)),
            pl.BlockSpec((1, NCHUNK, NB, 128), lambda i, l: (i, 0, 0, 0)),
        ],
        out_shape=[
            jax.ShapeDtypeStruct((N_LAYERS, N_EDGES, NSUP), jnp.float32),
            jax.ShapeDtypeStruct((NW, NCHUNK, NB, 128), jnp.int32),
        ],
    )(edge_attr, dstr, wabc, w14)


def _proj0_body(x_ref, wcf_ref, wf1_ref, wf2_ref, bf1_ref, bf2_ref,
                p_ref, g_ref):
    x = x_ref[...]
    p_ref[...] = jnp.dot(x, wcf_ref[...], preferred_element_type=jnp.float32)
    g_ref[...] = (
        jnp.tanh(jnp.dot(x, wf1_ref[...], preferred_element_type=jnp.float32)
                 + bf1_ref[...])
        * jnp.tanh(jnp.dot(x, wf2_ref[...], preferred_element_type=jnp.float32)
                   + bf2_ref[...]))


def _proj0(x, wcf, wf1, wf2, bf1, bf2):
    return pl.pallas_call(
        _proj0_body,
        out_shape=[
            jax.ShapeDtypeStruct((N_NODES, PDIM), jnp.float32),
            jax.ShapeDtypeStruct((N_NODES, NOUT), jnp.float32),
        ],
    )(x, wcf, wf1, wf2, bf1, bf2)


def _proj_mid_body(part_ref, gate_ref, bc_ref, wcf_ref, wf1_ref, wf2_ref,
                   bf1_ref, bf2_ref, p_ref, g_ref):
    out = jax.nn.relu(part_ref[0] + part_ref[1] + bc_ref[...])
    h = jnp.concatenate([out, gate_ref[...]], axis=1)
    p_ref[...] = jnp.dot(h, wcf_ref[...], preferred_element_type=jnp.float32)
    g_ref[...] = (
        jnp.tanh(jnp.dot(h, wf1_ref[...], preferred_element_type=jnp.float32)
                 + bf1_ref[...])
        * jnp.tanh(jnp.dot(h, wf2_ref[...], preferred_element_type=jnp.float32)
                   + bf2_ref[...]))


def _proj_mid(parts, gate, bc, wcf, wf1, wf2, bf1, bf2):
    return pl.pallas_call(
        _proj_mid_body,
        out_shape=[
            jax.ShapeDtypeStruct((N_NODES, PDIM), jnp.float32),
            jax.ShapeDtypeStruct((N_NODES, NOUT), jnp.float32),
        ],
    )(parts, gate, bc, wcf, wf1, wf2, bf1, bf2)


def _final_body(part_ref, gate_ref, bc_ref, batch_ref, fc1w_ref, fc1b_ref,
                fc2w_ref, fc2b_ref, out_ref):
    out = jax.nn.relu(part_ref[0] + part_ref[1] + bc_ref[...])
    h = jnp.concatenate([out, gate_ref[...]], axis=1)
    iota = lax.broadcasted_iota(jnp.int32, (N_GRAPHS, N_NODES), 0)
    onehot = jnp.where(iota == batch_ref[...], 1.0, 0.0)
    pooled = jnp.dot(onehot, h, preferred_element_type=jnp.float32)
    z = jax.nn.relu(
        jnp.dot(pooled, fc1w_ref[...], preferred_element_type=jnp.float32)
        + fc1b_ref[...])
    out_ref[...] = (jnp.dot(z, fc2w_ref[...],
                            preferred_element_type=jnp.float32)
                    + fc2b_ref[...])


def _final(parts, gate, bc, batch2d, fc1w, fc1b, fc2w, fc2b):
    return pl.pallas_call(
        _final_body,
        out_shape=jax.ShapeDtypeStruct((N_GRAPHS, 1), jnp.float32),
    )(parts, gate, bc, batch2d, fc1w, fc1b, fc2w, fc2b)


def kernel(x, edge_index, edge_attr, batch, params):
    layers = params["layers"]
    src = edge_index[0]
    dstr = edge_index[1].reshape(NW, EPW // 8, 8).astype(jnp.float32)

    wabc = jnp.stack([
        jnp.concatenate([lp["W11"], lp["W12"], lp["W13"]], axis=1)
        for lp in layers])
    w14 = jnp.stack([lp["W14"] for lp in layers])
    wcf = [jnp.transpose(lp["Wc"], (1, 0, 2)).reshape(-1, PDIM)
           for lp in layers]
    zero_hbm = jnp.zeros((FLAT // NS,), jnp.float32)

    ea_all, idx4 = _edge_mlp(edge_attr, dstr, wabc, w14)

    lp = layers[0]
    p, gate = _proj0(x, wcf[0],
                     lp["Wf11"], lp["Wf12"],
                     lp["bf11"].reshape(1, -1), lp["bf12"].reshape(1, -1))
    for l in range(N_LAYERS):
        parts = _sc_spectconv(l)(p, ea_all, src, idx4, zero_hbm)
        parts = parts.reshape(NC, N_PAD, NOUT)[:, :N_NODES, :]
        if l + 1 < N_LAYERS:
            nxt = layers[l + 1]
            p, gate = _proj_mid(parts, gate,
                                layers[l]["bc"].reshape(1, -1), wcf[l + 1],
                                nxt["Wf11"], nxt["Wf12"],
                                nxt["bf11"].reshape(1, -1),
                                nxt["bf12"].reshape(1, -1))
        else:
            out = _final(parts, gate, layers[l]["bc"].reshape(1, -1),
                         batch.reshape(1, -1).astype(jnp.int32),
                         params["fc1_W"], params["fc1_b"].reshape(1, -1),
                         params["fc2_W"], params["fc2_b"].reshape(1, -1))
    return out


# speedup vs baseline: 20.5871x; 1.0489x over previous
import functools

import jax
import jax.numpy as jnp
from jax import lax
from jax.experimental import pallas as pl
from jax.experimental.pallas import tpu as pltpu
from jax.experimental.pallas import tpu_sc as plsc

N_NODES = 10000
N_EDGES = 320000
N_GRAPHS = 16
N_LAYERS = 5
NSUP = 16
NOUT = 16
PDIM = NSUP * NOUT

NC = 2
NS = 16
NW = NC * NS
EPW = N_EDGES // NW
CHUNK = 128
NB = CHUNK * NOUT // 128
NCHUNK = 78
IDXG = 16
TAIL = EPW - NCHUNK * CHUNK
ACC_SLICE = 640
N_PAD = ACC_SLICE * NS
FLAT = N_PAD * NOUT

def _sc_spectconv_body(layer, p_hbm, ea_hbm, src_hbm, idx_hbm, zero_hbm,
                       out_hbm, src_v, idx_v, rows_v, ea_v, msgf_v, acc_sh,
                       lsem, gsem, ssem):
    cid = lax.axis_index("c")
    sid = lax.axis_index("s")
    wid = cid * NS + sid
    f0 = sid * (FLAT // NS)
    pltpu.sync_copy(zero_hbm, acc_sh.at[pl.ds(f0, FLAT // NS)])
    plsc.subcore_barrier()

    def fire_linear_at(base, g0, s):
        pltpu.async_copy(src_hbm.at[pl.ds(base, CHUNK)], src_v[s], lsem[s])
        pltpu.async_copy(ea_hbm.at[layer, pl.ds(base, CHUNK)], ea_v[s],
                         lsem[s])
        pltpu.async_copy(idx_hbm.at[wid, pl.ds(g0, CHUNK // IDXG)], idx_v[s],
                         lsem[s])

    def fire_linear(ci, s):
        fire_linear_at(wid * EPW + ci * CHUNK, ci * (CHUNK // IDXG), s)

    def drain_linear(s):
        pltpu.make_async_copy(src_hbm.at[pl.ds(0, CHUNK)], src_v[s],
                              lsem[s]).wait()
        pltpu.make_async_copy(ea_hbm.at[0, pl.ds(0, CHUNK)], ea_v[s],
                              lsem[s]).wait()
        pltpu.make_async_copy(idx_hbm.at[0, pl.ds(0, CHUNK // IDXG)],
                              idx_v[s], lsem[s]).wait()

    def fire_gather(s):
        pltpu.async_copy(p_hbm.at[src_v[s]], rows_v[s], gsem[s])

    def drain_gather(s):
        pltpu.make_async_copy(p_hbm.at[src_v[s]], rows_v[s], gsem[s]).wait()

    def fire_scatter(s, b0=0):
        descs = [pltpu.async_copy(msgf_v[s].at[pl.ds(b * 128, 128)],
                                  acc_sh.at[idx_v[s].at[b // 2, b % 2]],
                                  ssem[s], add=True)
                 for b in range(b0, NB)]
        for d in descs:
            d.wait()

    def drain_scatter(s):
        pass

    def compute(s):
        def edge_body(e, c2):
            av = ea_v[s][e, :]
            m = av[0] * rows_v[s][e, pl.ds(0, NOUT)]
            for i in range(1, NSUP):
                m = m + av[i] * rows_v[s][e, pl.ds(i * NOUT, NOUT)]
            msgf_v[s][pl.ds(e * NOUT, NOUT)] = m
            return c2
        lax.fori_loop(0, CHUNK, edge_body, 0, unroll=2)

    fire_linear(0, 0)
    drain_linear(0)
    fire_gather(0)

    def pair_body(g, carry):
        for s in range(2):
            ci = 2 * g + s
            if s == 0:
                @pl.when(g > 0)
                def _():
                    drain_scatter(1)
                    fire_linear(ci + 1, 1)
                    drain_linear(1)
                    drain_gather(0)
                    fire_gather(1)

                @pl.when(g == 0)
                def _():
                    fire_linear(ci + 1, 1)
                    drain_linear(1)
                    drain_gather(0)
                    fire_gather(1)
            else:
                drain_scatter(0)

                @pl.when(g < NCHUNK // 2 - 1)
                def _():
                    fire_linear(ci + 1, 0)
                    drain_linear(0)
                    drain_gather(1)
                    fire_gather(0)

                @pl.when(g == NCHUNK // 2 - 1)
                def _():
                    drain_gather(1)
            compute(s)
            fire_scatter(s)
        return carry

    lax.fori_loop(0, NCHUNK // 2, pair_body, 0)
    fire_linear_at(wid * EPW + EPW - CHUNK, (EPW - CHUNK) // IDXG, 0)
    drain_linear(0)
    fire_gather(0)
    drain_gather(0)
    compute(0)
    fire_scatter(0, b0=NB - TAIL * NOUT // 128)
    plsc.subcore_barrier()
    pltpu.sync_copy(acc_sh.at[pl.ds(f0, FLAT // NS)],
                    out_hbm.at[cid, pl.ds(f0, FLAT // NS)])


@functools.cache
def _sc_spectconv(layer):
    mesh = plsc.VectorSubcoreMesh(
        core_axis_name="c", subcore_axis_name="s",
        num_cores=NC, num_subcores=NS)
    return pl.kernel(
        functools.partial(_sc_spectconv_body, layer),
        out_type=jax.ShapeDtypeStruct((NC, FLAT), jnp.float32),
        mesh=mesh,
        scratch_types=[
            [pltpu.VMEM((CHUNK,), jnp.int32)] * 2,
            [pltpu.VMEM((CHUNK // IDXG, 2, 128), jnp.int32)] * 2,
            [pltpu.VMEM((CHUNK, PDIM), jnp.float32)] * 2,
            [pltpu.VMEM((CHUNK, NSUP), jnp.float32)] * 2,
            [pltpu.VMEM((CHUNK * NOUT,), jnp.float32)] * 2,
            pltpu.VMEM_SHARED((FLAT,), jnp.float32),
            [pltpu.SemaphoreType.DMA] * 2,
            [pltpu.SemaphoreType.DMA] * 2,
            [pltpu.SemaphoreType.DMA] * 2,
        ],
    )


def _edge_mlp_body(eattr_ref, dstr_ref, wabc_ref, w14_ref, out_ref, idx_ref):
    l = pl.program_id(1)
    ea = eattr_ref[...]
    t = jnp.dot(ea, wabc_ref[0], preferred_element_type=jnp.float32)
    tmp = (jax.nn.relu(t[:, :32])
           + jax.nn.relu(t[:, 32:64]) * jax.nn.relu(t[:, 64:96]))
    out_ref[0] = jax.nn.relu(
        jnp.dot(tmp, w14_ref[0], preferred_element_type=jnp.float32))

    @pl.when(l == 0)
    def _():
        sel = (lax.broadcasted_iota(jnp.int32, (8, 128), 1) // NOUT
               == lax.broadcasted_iota(jnp.int32, (8, 128), 0)).astype(jnp.float32)
        idxf = jnp.dot(dstr_ref[0], sel, preferred_element_type=jnp.float32,
                       precision=lax.Precision.HIGHEST)
        lanes = lax.broadcasted_iota(jnp.int32, (EPW // 8, 128), 1) % NOUT
        idx = (idxf * NOUT).astype(jnp.int32) + lanes
        idx_ref[...] = idx.reshape(1, EPW // IDXG, 2, 128)


def _edge_mlp(edge_attr, dstr, wabc, w14):
    return pl.pallas_call(
        _edge_mlp_body,
        grid=(NW, N_LAYERS),
        in_specs=[
            pl.BlockSpec((EPW, NSUP), lambda i, l: (i, 0)),
            pl.BlockSpec((1, EPW // 8, 8), lambda i, l: (i, 0, 0)),
            pl.BlockSpec((1, NSUP, 96), lambda i, l: (l, 0, 0)),
            pl.BlockSpec((1, 32, NSUP), lambda i, l: (l, 0, 0)),
        ],
        out_specs=[
            pl.BlockSpec((1, EPW, NSUP), lambda i, l: (l, i, 0)),
            pl.BlockSpec((1, EPW // IDXG, 2, 128),
                         lambda i, l: (i, 0, 0, 0)),
        ],
        out_shape=[
            jax.ShapeDtypeStruct((N_LAYERS, N_EDGES, NSUP), jnp.float32),
            jax.ShapeDtypeStruct((NW, EPW // IDXG, 2, 128), jnp.int32),
        ],
    )(edge_attr, dstr, wabc, w14)


def _proj0_body(x_ref, wcf_ref, wf1_ref, wf2_ref, bf1_ref, bf2_ref,
                p_ref, g_ref):
    x = x_ref[...]
    p_ref[...] = jnp.dot(x, wcf_ref[...], preferred_element_type=jnp.float32)
    g_ref[...] = (
        jnp.tanh(jnp.dot(x, wf1_ref[...], preferred_element_type=jnp.float32)
                 + bf1_ref[...])
        * jnp.tanh(jnp.dot(x, wf2_ref[...], preferred_element_type=jnp.float32)
                   + bf2_ref[...]))


def _proj0(x, wcf, wf1, wf2, bf1, bf2):
    return pl.pallas_call(
        _proj0_body,
        out_shape=[
            jax.ShapeDtypeStruct((N_NODES, PDIM), jnp.float32),
            jax.ShapeDtypeStruct((N_NODES, NOUT), jnp.float32),
        ],
    )(x, wcf, wf1, wf2, bf1, bf2)


def _proj_mid_body(part_ref, gate_ref, bc_ref, wcf_ref, wf1_ref, wf2_ref,
                   bf1_ref, bf2_ref, p_ref, g_ref):
    out = jax.nn.relu(part_ref[0] + part_ref[1] + bc_ref[...])
    h = jnp.concatenate([out, gate_ref[...]], axis=1)
    p_ref[...] = jnp.dot(h, wcf_ref[...], preferred_element_type=jnp.float32)
    g_ref[...] = (
        jnp.tanh(jnp.dot(h, wf1_ref[...], preferred_element_type=jnp.float32)
                 + bf1_ref[...])
        * jnp.tanh(jnp.dot(h, wf2_ref[...], preferred_element_type=jnp.float32)
                   + bf2_ref[...]))


def _proj_mid(parts, gate, bc, wcf, wf1, wf2, bf1, bf2):
    return pl.pallas_call(
        _proj_mid_body,
        out_shape=[
            jax.ShapeDtypeStruct((N_NODES, PDIM), jnp.float32),
            jax.ShapeDtypeStruct((N_NODES, NOUT), jnp.float32),
        ],
    )(parts, gate, bc, wcf, wf1, wf2, bf1, bf2)


def _final_body(part_ref, gate_ref, bc_ref, batch_ref, fc1w_ref, fc1b_ref,
                fc2w_ref, fc2b_ref, out_ref):
    out = jax.nn.relu(part_ref[0] + part_ref[1] + bc_ref[...])
    h = jnp.concatenate([out, gate_ref[...]], axis=1)
    iota = lax.broadcasted_iota(jnp.int32, (N_GRAPHS, N_NODES), 0)
    onehot = jnp.where(iota == batch_ref[...], 1.0, 0.0)
    pooled = jnp.dot(onehot, h, preferred_element_type=jnp.float32)
    z = jax.nn.relu(
        jnp.dot(pooled, fc1w_ref[...], preferred_element_type=jnp.float32)
        + fc1b_ref[...])
    out_ref[...] = (jnp.dot(z, fc2w_ref[...],
                            preferred_element_type=jnp.float32)
                    + fc2b_ref[...])


def _final(parts, gate, bc, batch2d, fc1w, fc1b, fc2w, fc2b):
    return pl.pallas_call(
        _final_body,
        out_shape=jax.ShapeDtypeStruct((N_GRAPHS, 1), jnp.float32),
    )(parts, gate, bc, batch2d, fc1w, fc1b, fc2w, fc2b)


def kernel(x, edge_index, edge_attr, batch, params):
    layers = params["layers"]
    src = edge_index[0]
    dstr = edge_index[1].reshape(NW, EPW // 8, 8).astype(jnp.float32)

    wabc = jnp.stack([
        jnp.concatenate([lp["W11"], lp["W12"], lp["W13"]], axis=1)
        for lp in layers])
    w14 = jnp.stack([lp["W14"] for lp in layers])
    wcf = [jnp.transpose(lp["Wc"], (1, 0, 2)).reshape(-1, PDIM)
           for lp in layers]
    zero_hbm = jnp.zeros((FLAT // NS,), jnp.float32)

    ea_all, idx4 = _edge_mlp(edge_attr, dstr, wabc, w14)

    lp = layers[0]
    p, gate = _proj0(x, wcf[0],
                     lp["Wf11"], lp["Wf12"],
                     lp["bf11"].reshape(1, -1), lp["bf12"].reshape(1, -1))
    for l in range(N_LAYERS):
        parts = _sc_spectconv(l)(p, ea_all, src, idx4, zero_hbm)
        parts = parts.reshape(NC, N_PAD, NOUT)[:, :N_NODES, :]
        if l + 1 < N_LAYERS:
            nxt = layers[l + 1]
            p, gate = _proj_mid(parts, gate,
                                layers[l]["bc"].reshape(1, -1), wcf[l + 1],
                                nxt["Wf11"], nxt["Wf12"],
                                nxt["bf11"].reshape(1, -1),
                                nxt["bf12"].reshape(1, -1))
        else:
            out = _final(parts, gate, layers[l]["bc"].reshape(1, -1),
                         batch.reshape(1, -1).astype(jnp.int32),
                         params["fc1_W"], params["fc1_b"].reshape(1, -1),
                         params["fc2_W"], params["fc2_b"].reshape(1, -1))
    return out
